# baseline (speedup 1.0000x reference)
import functools

import jax
import jax.numpy as jnp
from jax import lax
from jax.experimental import pallas as pl
from jax.experimental.pallas import tpu as pltpu
from jax.experimental.pallas import tpu_sc as plsc

_L = 16
_SC_R = 32
_TC_BLOCK = 1024


def _make_sc_kernel(rows_sc, rows_total, D, num_cores, num_subcores):
    NW = num_cores * num_subcores
    rows_per_w = rows_sc // NW
    R = _SC_R
    NCH = rows_per_w // R
    n_slices = D // _L
    mesh = plsc.VectorSubcoreMesh(core_axis_name="c", subcore_axis_name="s")

    @functools.partial(
        pl.kernel,
        mesh=mesh,
        out_type=jax.ShapeDtypeStruct((rows_sc, D), jnp.float32),
        scratch_types=[
            pltpu.VMEM((8,), jnp.int32),
            pltpu.VMEM((8, D), jnp.float32),
            pltpu.VMEM((R, D), jnp.float32),
            pltpu.VMEM((R, D), jnp.float32),
            pltpu.VMEM((R, D), jnp.float32),
            pltpu.SemaphoreType.DMA,
            pltpu.SemaphoreType.DMA,
            pltpu.SemaphoreType.DMA,
            pltpu.SemaphoreType.DMA,
            pltpu.SemaphoreType.DMA,
            pltpu.SemaphoreType.DMA,
            pltpu.SemaphoreType.DMA,
        ],
    )
    def sc_kernel(idx_hbm, emb_hbm, feat_hbm, out_hbm,
                  idx_v, row_v, b0, b1, b2,
                  sem_row, si0, si1, si2, so0, so1, so2):
        c = lax.axis_index("c")
        s = lax.axis_index("s")
        wid = s * num_cores + c
        base = wid * rows_per_w

        pltpu.sync_copy(idx_hbm, idx_v)
        pltpu.make_async_copy(emb_hbm.at[idx_v], row_v, sem_row).start()

        bufs = (b0, b1, b2)
        isems = (si0, si1, si2)
        osems = (so0, so1, so2)

        def in_cp(g, b):
            return pltpu.make_async_copy(
                feat_hbm.at[pl.ds(base + g * R, R)], bufs[b], isems[b])

        def out_cp(g, b):
            return pltpu.make_async_copy(
                bufs[b], out_hbm.at[pl.ds(base + g * R, R)], osems[b])

        in_cp(0, 0).start()
        in_cp(1, 1).start()
        pltpu.make_async_copy(emb_hbm.at[idx_v], row_v, sem_row).wait()

        def add_chunk(buf):
            for q in range(n_slices // 16):
                held = [row_v[0, pl.ds((q * 16 + j) * _L, _L)]
                        for j in range(16)]

                @plsc.parallel_loop(0, R, unroll=2)
                def row_body(r):
                    for j in range(16):
                        off = (q * 16 + j) * _L
                        buf[r, pl.ds(off, _L)] = (
                            buf[r, pl.ds(off, _L)] + held[j])

        def do_chunk(g, b):
            in_cp(g, b).wait()
            add_chunk(bufs[b])
            out_cp(g, b).start()

            @pl.when(g + 2 < NCH)
            def _():
                @pl.when(g >= 1)
                def _():
                    out_cp(g - 1, (b + 2) % 3).wait()

                in_cp(g + 2, (b + 2) % 3).start()

        ntriples = NCH // 3

        def step(t, carry):
            for b in range(3):
                do_chunk(3 * t + b, b)
            return carry

        lax.fori_loop(0, ntriples, step, 0)
        for g in range(3 * ntriples, NCH):
            do_chunk(g, g % 3)
        for g in range(max(0, NCH - 3), NCH):
            out_cp(g, g % 3).wait()

    return sc_kernel


def _tc_body(idx_ref, emb_ref, feat_ref, out_ref):
    row = emb_ref[pl.ds(idx_ref[0], 1), :]
    out_ref[...] = feat_ref[...] + row


def _tc_add(idx, type_embedding, flat, row_start, rows_total, D):
    nblk = (rows_total - row_start) // _TC_BLOCK
    blk0 = row_start // _TC_BLOCK
    return pl.pallas_call(
        _tc_body,
        grid_spec=pltpu.PrefetchScalarGridSpec(
            num_scalar_prefetch=1,
            grid=(nblk,),
            in_specs=[
                pl.BlockSpec(type_embedding.shape, lambda i, idx: (0, 0)),
                pl.BlockSpec((_TC_BLOCK, D), lambda i, idx: (i + blk0, 0)),
            ],
            out_specs=pl.BlockSpec((_TC_BLOCK, D), lambda i, idx: (i + blk0, 0)),
        ),
        out_shape=jax.ShapeDtypeStruct((rows_total, D), jnp.float32),
    )(idx, type_embedding, flat)


def kernel(features, view_type_id, type_embedding):
    squeeze = False
    if features.ndim == 2:
        features = features[None, :, :]
        squeeze = True
    B, N, D = features.shape
    rows = B * N
    flat = features.reshape(rows, D)
    idx = jnp.full((8,), view_type_id, dtype=jnp.int32)

    info = plsc.get_sparse_core_info()
    NW = info.num_cores * info.num_subcores

    sc_rows = rows // 4
    sc_unit = NW * _SC_R
    sc_rows -= sc_rows % sc_unit
    if sc_rows == 0 or (rows - sc_rows) % _TC_BLOCK != 0:
        sc_rows = rows

    sc = _make_sc_kernel(sc_rows, rows, D, info.num_cores, info.num_subcores)
    sc_out = sc(idx, type_embedding, flat)

    if sc_rows == rows:
        out = sc_out
    else:
        tc_out = _tc_add(idx[:1], type_embedding, flat, sc_rows, rows, D)
        out = lax.dynamic_update_slice(tc_out, sc_out, (0, 0))

    out = out.reshape(B, N, D)
    if squeeze:
        return out[0]
    return out


# speedup vs baseline: 1.0284x; 1.0284x over previous
import functools

import jax
import jax.numpy as jnp
from jax import lax
from jax.experimental import pallas as pl
from jax.experimental.pallas import tpu as pltpu
from jax.experimental.pallas import tpu_sc as plsc

_L = 16


def _make_sc_kernel(rows, D, num_cores, num_subcores):
    NW = num_cores * num_subcores
    rows_per_w = rows // NW
    R = 16
    NCH = rows_per_w // R
    n_slices = D // _L
    mesh = plsc.VectorSubcoreMesh(core_axis_name="c", subcore_axis_name="s")

    @functools.partial(
        pl.kernel,
        mesh=mesh,
        out_type=jax.ShapeDtypeStruct((rows, D), jnp.float32),
        scratch_types=[
            pltpu.VMEM((8,), jnp.int32),
            pltpu.VMEM((8, D), jnp.float32),
            pltpu.VMEM((R, D), jnp.float32),
            pltpu.VMEM((R, D), jnp.float32),
            pltpu.VMEM((R, D), jnp.float32),
            pltpu.VMEM((R, D), jnp.float32),
            pltpu.SemaphoreType.DMA,
            pltpu.SemaphoreType.DMA,
            pltpu.SemaphoreType.DMA,
            pltpu.SemaphoreType.DMA,
            pltpu.SemaphoreType.DMA,
        ],
    )
    def sc_kernel(idx_hbm, emb_hbm, feat_hbm, out_hbm,
                  idx_v, row_v, in0, in1, ob0, ob1,
                  sem_row, si0, si1, so0, so1):
        c = lax.axis_index("c")
        s = lax.axis_index("s")
        wid = s * num_cores + c
        base = wid * rows_per_w

        pltpu.sync_copy(idx_hbm, idx_v)
        pltpu.make_async_copy(emb_hbm.at[idx_v], row_v, sem_row).start()

        in_bufs = (in0, in1)
        out_bufs = (ob0, ob1)
        isems = (si0, si1)
        osems = (so0, so1)

        def in_cp(g, b):
            return pltpu.make_async_copy(
                feat_hbm.at[pl.ds(base + g * R, R)], in_bufs[b], isems[b])

        def out_cp(g, b):
            return pltpu.make_async_copy(
                out_bufs[b], out_hbm.at[pl.ds(base + g * R, R)], osems[b])

        in_cp(0, 0).start()
        in_cp(1, 1).start()
        pltpu.make_async_copy(emb_hbm.at[idx_v], row_v, sem_row).wait()

        def add_chunk(src, dst):
            for q in range(n_slices // 16):
                held = [row_v[0, pl.ds((q * 16 + j) * _L, _L)]
                        for j in range(16)]

                @plsc.parallel_loop(0, R, unroll=2)
                def row_body(r):
                    for j in range(16):
                        off = (q * 16 + j) * _L
                        dst[r, pl.ds(off, _L)] = (
                            src[r, pl.ds(off, _L)] + held[j])

        def step(t, carry):
            for b in range(2):
                g = 2 * t + b
                in_cp(g, b).wait()

                @pl.when(g >= 2)
                def _():
                    out_cp(g - 2, b).wait()

                add_chunk(in_bufs[b], out_bufs[b])

                @pl.when(g + 2 < NCH)
                def _():
                    in_cp(g + 2, b).start()

                out_cp(g, b).start()
            return carry

        lax.fori_loop(0, NCH // 2, step, 0)
        out_cp(NCH - 2, 0).wait()
        out_cp(NCH - 1, 1).wait()

    return sc_kernel


def kernel(features, view_type_id, type_embedding):
    squeeze = False
    if features.ndim == 2:
        features = features[None, :, :]
        squeeze = True
    B, N, D = features.shape
    rows = B * N
    flat = features.reshape(rows, D)
    idx = jnp.full((8,), view_type_id, dtype=jnp.int32)

    info = plsc.get_sparse_core_info()
    sc = _make_sc_kernel(rows, D, info.num_cores, info.num_subcores)
    out = sc(idx, type_embedding, flat)

    out = out.reshape(B, N, D)
    if squeeze:
        return out[0]
    return out
